# Initial kernel scaffold; baseline (speedup 1.0000x reference)
import jax
import jax.numpy as jnp
from jax.experimental import pallas as pl


def kernel(x, embed):
    raise NotImplementedError("write your pallas kernel here")


# speedup vs baseline: 1.5524x; 1.5524x over previous
import jax
import jax.numpy as jnp
from jax.experimental import pallas as pl
from jax.experimental.pallas import tpu as pltpu

_DIM = 64
_K = 1024
_BLOCK_M = 2048


def _vq_body(x_ref, e_ref, q_ref, i_ref):
    xb = x_ref[...]
    e = e_ref[...]
    a = jnp.sum(xb * xb, axis=1, keepdims=True)
    prod = jax.lax.dot_general(
        xb, e, (((1,), (1,)), ((), ())),
        preferred_element_type=jnp.float32)
    b = jnp.sum(e * e, axis=1)[None, :]
    dist = -(a - 2.0 * prod + b)
    idx = jnp.argmax(dist, axis=1).astype(jnp.int32)
    i_ref[...] = idx
    onehot = (jax.lax.broadcasted_iota(jnp.int32, dist.shape, 1)
              == idx[:, None]).astype(jnp.float32)
    q_ref[...] = jax.lax.dot_general(
        onehot, e, (((1,), (0,)), ((), ())),
        preferred_element_type=jnp.float32)


def kernel(x, embed):
    shape = x.shape
    flat = x.reshape(-1, shape[-1])
    m = flat.shape[0]
    grid = (m // _BLOCK_M,)
    quant, idx = pl.pallas_call(
        _vq_body,
        grid=grid,
        in_specs=[
            pl.BlockSpec((_BLOCK_M, _DIM), lambda i: (i, 0)),
            pl.BlockSpec((_K, _DIM), lambda i: (0, 0)),
        ],
        out_specs=[
            pl.BlockSpec((_BLOCK_M, _DIM), lambda i: (i, 0)),
            pl.BlockSpec((_BLOCK_M,), lambda i: (i,)),
        ],
        out_shape=[
            jax.ShapeDtypeStruct((m, _DIM), jnp.float32),
            jax.ShapeDtypeStruct((m,), jnp.int32),
        ],
        compiler_params=pltpu.CompilerParams(
            dimension_semantics=("parallel",)),
    )(flat, embed)
    return quant.reshape(shape), idx.reshape(shape[:-1])


# speedup vs baseline: 1.5936x; 1.0266x over previous
import jax
import jax.numpy as jnp
from jax.experimental import pallas as pl
from jax.experimental.pallas import tpu as pltpu

_DIM = 64
_K = 1024
_BLOCK_M = 1024


def _vq_body(x_ref, e_ref, em2_ref, q_ref, i_ref):
    xb = x_ref[...]
    e = e_ref[...]
    em2 = em2_ref[...]
    a = jnp.sum(xb * xb, axis=1, keepdims=True)
    prod = jax.lax.dot_general(
        xb, em2, (((1,), (1,)), ((), ())),
        preferred_element_type=jnp.float32)
    b = jnp.sum(e * e, axis=1)[None, :]
    dist = (a + prod) + b
    idx = jnp.argmin(dist, axis=1).astype(jnp.int32)
    i_ref[...] = idx
    onehot = (jax.lax.broadcasted_iota(jnp.int32, dist.shape, 1)
              == idx[:, None]).astype(jnp.float32)
    q_ref[...] = jax.lax.dot_general(
        onehot, e, (((1,), (0,)), ((), ())),
        preferred_element_type=jnp.float32)


def kernel(x, embed):
    shape = x.shape
    flat = x.reshape(-1, shape[-1])
    m = flat.shape[0]
    grid = (m // _BLOCK_M,)
    quant, idx = pl.pallas_call(
        _vq_body,
        grid=grid,
        in_specs=[
            pl.BlockSpec((_BLOCK_M, _DIM), lambda i: (i, 0)),
            pl.BlockSpec((_K, _DIM), lambda i: (0, 0)),
            pl.BlockSpec((_K, _DIM), lambda i: (0, 0)),
        ],
        out_specs=[
            pl.BlockSpec((_BLOCK_M, _DIM), lambda i: (i, 0)),
            pl.BlockSpec((_BLOCK_M,), lambda i: (i,)),
        ],
        out_shape=[
            jax.ShapeDtypeStruct((m, _DIM), jnp.float32),
            jax.ShapeDtypeStruct((m,), jnp.int32),
        ],
        compiler_params=pltpu.CompilerParams(
            dimension_semantics=("parallel",)),
    )(flat, embed, -2.0 * embed)
    return quant.reshape(shape), idx.reshape(shape[:-1])


# speedup vs baseline: 1.6811x; 1.0549x over previous
import jax
import jax.numpy as jnp
from jax.experimental import pallas as pl
from jax.experimental.pallas import tpu as pltpu

_DIM = 64
_K = 1024
_BLOCK_M = 2048


def _vq_body(x_ref, e_ref, em2_ref, q_ref, i_ref):
    xb = x_ref[...]
    e = e_ref[...]
    em2 = em2_ref[...]
    a = jnp.sum(xb * xb, axis=1, keepdims=True)
    prod = jax.lax.dot_general(
        xb, em2, (((1,), (1,)), ((), ())),
        preferred_element_type=jnp.float32)
    b = jnp.sum(e * e, axis=1)[None, :]
    dist = (a + prod) + b
    idx = jnp.argmin(dist, axis=1).astype(jnp.int32)
    i_ref[...] = idx
    onehot = (jax.lax.broadcasted_iota(jnp.int32, dist.shape, 1)
              == idx[:, None]).astype(jnp.float32)
    q_ref[...] = jax.lax.dot_general(
        onehot, e, (((1,), (0,)), ((), ())),
        preferred_element_type=jnp.float32)


def kernel(x, embed):
    shape = x.shape
    flat = x.reshape(-1, shape[-1])
    m = flat.shape[0]
    grid = (m // _BLOCK_M,)
    quant, idx = pl.pallas_call(
        _vq_body,
        grid=grid,
        in_specs=[
            pl.BlockSpec((_BLOCK_M, _DIM), lambda i: (i, 0)),
            pl.BlockSpec((_K, _DIM), lambda i: (0, 0)),
            pl.BlockSpec((_K, _DIM), lambda i: (0, 0)),
        ],
        out_specs=[
            pl.BlockSpec((_BLOCK_M, _DIM), lambda i: (i, 0)),
            pl.BlockSpec((_BLOCK_M,), lambda i: (i,)),
        ],
        out_shape=[
            jax.ShapeDtypeStruct((m, _DIM), jnp.float32),
            jax.ShapeDtypeStruct((m,), jnp.int32),
        ],
        compiler_params=pltpu.CompilerParams(
            dimension_semantics=("parallel",)),
    )(flat, embed, -2.0 * embed)
    return quant.reshape(shape), idx.reshape(shape[:-1])
